# baseline (speedup 1.0000x reference)
import functools

import jax
import jax.numpy as jnp
from jax import lax
from jax.experimental import pallas as pl
from jax.experimental.pallas import tpu as pltpu
from jax.experimental.pallas import tpu_sc as plsc

B = 16384
L = 50
D = 32
V = 1000000

BLV = 8192
PGRID = 123
NPAIR = PGRID * BLV
RPV = 8
NROW8 = NPAIR // RPV

NC, NS = 2, 16
NW = NC * NS
BAGS_W = B // NW
CB = 16
STEPS = BAGS_W // CB
IPC = CB * L
GN = 80
NG = IPC // GN
NPL = 6


def _take(vec, idx):
    return vec.at[idx].get(mode="promise_in_bounds")


def _pack_kernel(wt_ref, lo_ref, hi_ref):
    w = wt_ref[...]
    w3 = w.reshape(32, BLV // 128, 128)
    neg = (w3 < 0.0)
    nz = (w3 != 0.0)
    d = lax.broadcasted_iota(jnp.int32, (32, BLV // 128, 128), 0)
    t = jnp.where(d < 16, d, d - 16)
    negbit = jnp.where(neg, jnp.int32(1) << (2 * t), 0)
    nzbit = jnp.where(nz, jnp.int32(1) << (2 * t + 1), 0)
    contrib = negbit | nzbit
    lo_ref[...] = jnp.sum(jnp.where(d < 16, contrib, 0), axis=0)
    hi_ref[...] = jnp.sum(jnp.where(d < 16, 0, contrib), axis=0)


def _bag_kernel(x_hbm, p_hbm, out_hbm, idx_v, idx8_v, rows_v, out_v, sem0, sem1):
    cid = lax.axis_index("c")
    sid = lax.axis_index("s")
    wid = sid * NC + cid
    base_bag = wid * BAGS_W
    sems = (sem0, sem1)

    lane = lax.iota(jnp.int32, 16)
    bag_lane = lane >> 1
    col_lane = lane & 1
    lane_masks = [bag_lane == j for j in range(8)]
    mneg = (jnp.uint32(1) << (2 * lane).astype(jnp.uint32)).astype(jnp.uint32)
    mnz = mneg + mneg
    zero16 = lane * 0
    one_f = zero16.astype(jnp.float32) + 1.0

    def fire(s, buf):
        bag0 = base_bag + s * CB
        f0 = pl.multiple_of(bag0 * L, 8)
        pltpu.sync_copy(x_hbm.at[pl.ds(f0, IPC)], idx_v.at[buf, pl.ds(0, IPC)])

        def sh(j, c):
            for q in range(GN // 16):
                off = pl.multiple_of(j * GN + q * 16, 8)
                idx8_v[buf, j, pl.ds(q * 16, 16)] = (
                    idx_v[buf, pl.ds(off, 16)] >> 3
                )
            return c

        lax.fori_loop(0, NG, sh, 0)
        for j in range(NG):
            pltpu.async_copy(
                p_hbm.at[idx8_v.at[buf, j]],
                rows_v.at[buf, pl.ds(j * GN, GN)],
                sems[buf],
            )

    def drain(buf):
        pltpu.make_async_copy(
            p_hbm.at[pl.ds(0, IPC)], rows_v.at[buf], sems[buf]
        ).wait()

    def compute(s, buf):
        bag0 = base_bag + s * CB
        for g in range(CB // 8):
            pos_g = 8 * g

            def row_body(r, planes):
                pos0 = r * CB + pos_g
                subs = idx_v[buf, pl.ds(pos0, 16)]
                up = _take(subs, bag_lane)
                lanesel = (up & 7) * 2 + col_lane
                w = jnp.zeros((16,), jnp.int32)
                for j in range(8):
                    wj = rows_v[buf, pos0 + j, pl.ds(0, 16)]
                    w = jnp.where(lane_masks[j], _take(wj, lanesel), w)
                out = []
                c = w
                for k in range(NPL):
                    out.append(planes[k] ^ c)
                    c = planes[k] & c
                return tuple(out)

            zero = jnp.zeros((16,), jnp.int32)
            planes = lax.fori_loop(0, L, row_body, (zero,) * NPL)

            def b_body(bl, carry):
                for h in range(2):
                    li = zero16 + (2 * bl + h)
                    negc = zero16
                    nzc = zero16
                    for k in range(NPL):
                        w = _take(planes[k], li).astype(jnp.uint32)
                        negc = negc + jnp.where((w & mneg) != 0, 1 << k, 0)
                        nzc = nzc + jnp.where((w & mnz) != 0, 1 << k, 0)
                    sv = (nzc - 2 * negc).astype(jnp.float32)
                    sgn = jnp.where(
                        sv > 0.0, one_f, jnp.where(sv < 0.0, -one_f, 0.0 * one_f)
                    )
                    out_v[pos_g + bl, pl.ds(16 * h, 16)] = sgn
                return carry

            lax.fori_loop(0, 8, b_body, 0)
        pltpu.sync_copy(out_v, out_hbm.at[pl.ds(pl.multiple_of(bag0, 8), CB)])

    fire(0, 0)

    def body(h, carry):
        s0 = 2 * h
        fire(s0 + 1, 1)
        drain(0)
        compute(s0, 0)

        @pl.when(s0 + 2 < STEPS)
        def _():
            fire(s0 + 2, 0)

        drain(1)
        compute(s0 + 1, 1)
        return carry

    lax.fori_loop(0, STEPS // 2, body, 0)


@jax.jit
def kernel(x, weight):
    lo, hi = pl.pallas_call(
        _pack_kernel,
        grid=(PGRID,),
        in_specs=[pl.BlockSpec((32, BLV), lambda i: (0, i))],
        out_specs=[
            pl.BlockSpec((BLV // 128, 128), lambda i: (i, 0)),
            pl.BlockSpec((BLV // 128, 128), lambda i: (i, 0)),
        ],
        out_shape=[
            jax.ShapeDtypeStruct((NPAIR // 128, 128), jnp.int32),
            jax.ShapeDtypeStruct((NPAIR // 128, 128), jnp.int32),
        ],
    )(weight.T)
    pairs = jnp.stack([lo.reshape(NPAIR), hi.reshape(NPAIR)], axis=1)
    p16 = pairs.reshape(NROW8, 16)

    xp = (
        x.astype(jnp.int32)
        .reshape(B // CB, CB, L)
        .transpose(0, 2, 1)
        .reshape(B * L)
    )

    mesh = plsc.VectorSubcoreMesh(core_axis_name="c", subcore_axis_name="s")
    f = pl.kernel(
        _bag_kernel,
        out_type=jax.ShapeDtypeStruct((B, D), jnp.float32),
        mesh=mesh,
        scratch_types=[
            pltpu.VMEM((2, IPC + 16), jnp.int32),
            pltpu.VMEM((2, NG, GN), jnp.int32),
            pltpu.VMEM((2, IPC, 16), jnp.int32),
            pltpu.VMEM((CB, D), jnp.float32),
            pltpu.SemaphoreType.DMA,
            pltpu.SemaphoreType.DMA,
        ],
        compiler_params=pltpu.CompilerParams(use_tc_tiling_on_sc=False),
    )
    return f(xp, p16)


# speedup vs baseline: 2.9396x; 2.9396x over previous
import functools

import jax
import jax.numpy as jnp
from jax import lax
from jax.experimental import pallas as pl
from jax.experimental.pallas import tpu as pltpu
from jax.experimental.pallas import tpu_sc as plsc

B = 16384
L = 50
D = 32
V = 1000000

BLV = 8192
PGRID = 123
NPAIR = PGRID * BLV
RPV = 8
NROW8 = NPAIR // RPV

NC, NS = 2, 16
NW = NC * NS
BAGS_W = B // NW
CB = 16
STEPS = BAGS_W // CB
IPC = CB * L
GN = 80
NG = IPC // GN
NPL = 6


def _take(vec, idx):
    return vec.at[idx].get(mode="promise_in_bounds")


def _pack_kernel(wt_ref, lo_ref, hi_ref):
    w = wt_ref[...]
    w3 = w.reshape(32, BLV // 128, 128)
    neg = (w3 < 0.0)
    nz = (w3 != 0.0)
    d = lax.broadcasted_iota(jnp.int32, (32, BLV // 128, 128), 0)
    t = jnp.where(d < 16, d, d - 16)
    negbit = jnp.where(neg, jnp.int32(1) << (2 * t), 0)
    nzbit = jnp.where(nz, jnp.int32(1) << (2 * t + 1), 0)
    contrib = negbit | nzbit
    lo_ref[...] = jnp.sum(jnp.where(d < 16, contrib, 0), axis=0)
    hi_ref[...] = jnp.sum(jnp.where(d < 16, 0, contrib), axis=0)


def _bag_kernel(x_hbm, p_hbm, out_hbm, idx_v, idx8_v, rows_v, out_v, sem0, sem1):
    cid = lax.axis_index("c")
    sid = lax.axis_index("s")
    wid = sid * NC + cid
    base_bag = wid * BAGS_W
    sems = (sem0, sem1)

    lane = lax.iota(jnp.int32, 16)
    bag_lane = lane >> 1
    col_lane = lane & 1
    lane_masks = [bag_lane == j for j in range(8)]
    mneg = (jnp.uint32(1) << (2 * lane).astype(jnp.uint32)).astype(jnp.uint32)
    mnz = mneg + mneg
    zero16 = lane * 0
    one_f = zero16.astype(jnp.float32) + 1.0

    def fire(s, buf):
        bag0 = base_bag + s * CB
        f0 = pl.multiple_of(bag0 * L, 8)
        pltpu.sync_copy(x_hbm.at[pl.ds(f0, IPC)], idx_v.at[buf, pl.ds(0, IPC)])

        def sh(j, c):
            for q in range(GN // 16):
                off = pl.multiple_of(j * GN + q * 16, 8)
                idx8_v[buf, j, pl.ds(q * 16, 16)] = (
                    idx_v[buf, pl.ds(off, 16)] >> 3
                )
            return c

        lax.fori_loop(0, NG, sh, 0)
        for j in range(NG):
            pltpu.async_copy(
                p_hbm.at[idx8_v.at[buf, j]],
                rows_v.at[buf, pl.ds(j * GN, GN)],
                sems[buf],
            )

    def drain(buf):
        pltpu.make_async_copy(
            p_hbm.at[pl.ds(0, IPC)], rows_v.at[buf], sems[buf]
        ).wait()

    def compute(s, buf):
        bag0 = base_bag + s * CB
        for g in range(CB // 8):
            pos_g = 8 * g

            def row_body(r, planes):
                pos0 = r * CB + pos_g
                subs = idx_v[buf, pl.ds(pos0, 16)]
                up = _take(subs, bag_lane)
                lanesel = (up & 7) * 2 + col_lane
                w = jnp.zeros((16,), jnp.int32)
                for j in range(8):
                    wj = rows_v[buf, pos0 + j, pl.ds(0, 16)]
                    w = jnp.where(lane_masks[j], _take(wj, lanesel), w)
                out = []
                c = w
                for k in range(NPL):
                    out.append(planes[k] ^ c)
                    c = planes[k] & c
                return tuple(out)

            zero = jnp.zeros((16,), jnp.int32)
            planes = lax.fori_loop(0, L, row_body, (zero,) * NPL)

            def b_body(bl, carry):
                for h in range(2):
                    li = zero16 + (2 * bl + h)
                    negc = zero16
                    nzc = zero16
                    for k in range(NPL):
                        w = _take(planes[k], li).astype(jnp.uint32)
                        negc = negc + jnp.where((w & mneg) != 0, 1 << k, 0)
                        nzc = nzc + jnp.where((w & mnz) != 0, 1 << k, 0)
                    sv = (nzc - 2 * negc).astype(jnp.float32)
                    sgn = jnp.where(
                        sv > 0.0, one_f, jnp.where(sv < 0.0, -one_f, 0.0 * one_f)
                    )
                    out_v[pos_g + bl, pl.ds(16 * h, 16)] = sgn
                return carry

            lax.fori_loop(0, 8, b_body, 0)
        pltpu.sync_copy(out_v, out_hbm.at[pl.ds(pl.multiple_of(bag0, 8), CB)])

    fire(0, 0)

    def body(h, carry):
        s0 = 2 * h
        fire(s0 + 1, 1)
        drain(0)
        compute(s0, 0)

        @pl.when(s0 + 2 < STEPS)
        def _():
            fire(s0 + 2, 0)

        drain(1)
        compute(s0 + 1, 1)
        return carry

    lax.fori_loop(0, STEPS // 2, body, 0)


@jax.jit
def kernel(x, weight):
    lo, hi = pl.pallas_call(
        _pack_kernel,
        grid=(PGRID,),
        in_specs=[pl.BlockSpec((32, BLV), lambda i: (0, i))],
        out_specs=[
            pl.BlockSpec((BLV // 128, 128), lambda i: (i, 0)),
            pl.BlockSpec((BLV // 128, 128), lambda i: (i, 0)),
        ],
        out_shape=[
            jax.ShapeDtypeStruct((NPAIR // 128, 128), jnp.int32),
            jax.ShapeDtypeStruct((NPAIR // 128, 128), jnp.int32),
        ],
    )(weight.T)
    z = jnp.int32(0)
    lo_up = lax.pad(lo.reshape(NPAIR), z, [(0, 1, 1)])
    hi_up = lax.pad(hi.reshape(NPAIR), z, [(1, 0, 1)])
    p16 = (lo_up + hi_up).reshape(NROW8, 16)

    xp = (
        x.astype(jnp.int32)
        .reshape(B // CB, CB, L)
        .transpose(0, 2, 1)
        .reshape(B * L)
    )

    mesh = plsc.VectorSubcoreMesh(core_axis_name="c", subcore_axis_name="s")
    f = pl.kernel(
        _bag_kernel,
        out_type=jax.ShapeDtypeStruct((B, D), jnp.float32),
        mesh=mesh,
        scratch_types=[
            pltpu.VMEM((2, IPC + 16), jnp.int32),
            pltpu.VMEM((2, NG, GN), jnp.int32),
            pltpu.VMEM((2, IPC, 16), jnp.int32),
            pltpu.VMEM((CB, D), jnp.float32),
            pltpu.SemaphoreType.DMA,
            pltpu.SemaphoreType.DMA,
        ],
        compiler_params=pltpu.CompilerParams(use_tc_tiling_on_sc=False),
    )
    return f(xp, p16)
